# Initial kernel scaffold; baseline (speedup 1.0000x reference)
import jax
import jax.numpy as jnp
from jax.experimental import pallas as pl


def kernel(output, target):
    raise NotImplementedError("write your pallas kernel here")


# speedup vs baseline: 161.0076x; 161.0076x over previous
import jax
import jax.numpy as jnp
from jax.experimental import pallas as pl
from jax.experimental.pallas import tpu as pltpu

_THRESHOLD = 0.05
_IOU_THRESHOLD = 0.3
_LEN_THRESHOLD = 10.0

_N = 32768
_LOGN = 15
_ROWS = 8
_NEG = -3.0
_FN = float(_N)


def _lane():
    return jax.lax.broadcasted_iota(jnp.int32, (_ROWS, _N), 1)


def _dshl(x, s, fill):
    if x.dtype == jnp.bool_:
        return _dshl(x.astype(jnp.int32), s, jnp.int32(fill)) != 0
    r = pltpu.roll(x, s, axis=1)
    return jnp.where(_lane() < s, jnp.full_like(x, fill), r)


def _dshr(x, s, fill):
    if x.dtype == jnp.bool_:
        return _dshr(x.astype(jnp.int32), s, jnp.int32(fill)) != 0
    r = pltpu.roll(x, _N - s, axis=1)
    return jnp.where(_lane() >= _N - s, jnp.full_like(x, fill), r)


def _geometry_loop(v1, v2, v3, v4):
    def step(k, carry):
        s = jnp.int32(1) << k
        v1[...] = jnp.maximum(v1[...], _dshl(v1[...], s, -1.0))
        v2[...] = jnp.maximum(v2[...], _dshl(v2[...], s, -1.0))
        v3[...] = jnp.minimum(v3[...], _dshr(v3[...], s, _FN))
        v4[...] = jnp.minimum(v4[...], _dshr(v4[...], s, _FN))
        return carry

    jax.lax.fori_loop(0, _LOGN, step, 0)


def _seg_fwd_loop(va, fa, vb, fb):
    def step(k, carry):
        s = jnp.int32(1) << k
        for v, f in ((va, fa), (vb, fb)):
            vv, ff = v[...], f[...]
            vsh = _dshl(vv, s, _NEG)
            fsh = _dshl(ff, s, 0.0)
            v[...] = jnp.where(ff > 0.5, vv, jnp.maximum(vv, vsh))
            f[...] = jnp.maximum(ff, fsh)
        return carry

    jax.lax.fori_loop(0, _LOGN, step, 0)


def _seg_bwd_loop(va, fa, vb, fb):
    def step(k, carry):
        s = jnp.int32(1) << k
        for v, h in ((va, fa), (vb, fb)):
            vv, hh = v[...], h[...]
            vsh = _dshr(vv, s, _NEG)
            hsh = _dshr(hh, s, 1.0)
            v[...] = jnp.where(hh > 0.5, vv, jnp.maximum(vv, vsh))
            h[...] = jnp.maximum(hh, hsh)
        return carry

    jax.lax.fori_loop(0, _LOGN, step, 0)


def _body(out_ref, tgt_ref, res_ref, v1, v2, v3, v4, f1, f2, riou, rsp, rst,
          acc_ref):
    step = pl.program_id(0)

    @pl.when(step == 0)
    def _init():
        acc_ref[0] = 0.0
        acc_ref[1] = 0.0
        acc_ref[2] = 0.0

    p = out_ref[...] >= _THRESHOLD
    t = tgt_ref[...] != 0

    pos = _lane().astype(jnp.float32)
    start_p = jnp.logical_and(p, jnp.logical_not(_dshl(p, 1, False)))
    start_t = jnp.logical_and(t, jnp.logical_not(_dshl(t, 1, False)))

    v1[...] = jnp.where(start_p, pos, -1.0)
    v2[...] = jnp.where(start_t, pos, -1.0)
    v3[...] = jnp.where(p, _FN, pos)
    v4[...] = jnp.where(t, _FN, pos)
    _geometry_loop(v1, v2, v3, v4)
    lsp, lst, nzp, nzt = v1[...], v2[...], v3[...], v4[...]

    both = jnp.logical_and(p, t)
    pairstart = jnp.logical_and(
        both,
        jnp.logical_or(
            jnp.logical_not(_dshl(both, 1, False)),
            jnp.logical_or(start_p, start_t),
        ),
    )

    la = nzp - lsp
    lb = nzt - lst
    inter = jnp.minimum(nzp, nzt) - pos
    iou_raw = inter / (la + lb - inter)
    valid = jnp.logical_and(pairstart, la >= _LEN_THRESHOLD)
    iou = jnp.where(valid, iou_raw, -1.0)
    a_flag = jnp.logical_and(start_p, la >= _LEN_THRESHOLD)

    sp_f = jnp.where(start_p, 1.0, 0.0)
    st_f = jnp.where(start_t, 1.0, 0.0)
    riou[...] = iou
    rsp[...] = sp_f
    rst[...] = st_f

    v1[...] = iou
    f1[...] = sp_f
    v2[...] = iou
    f2[...] = st_f
    _seg_fwd_loop(v1, f1, v2, f2)
    fr = jnp.where(start_p, _NEG, _dshl(v1[...], 1, _NEG))
    fc = jnp.where(start_t, _NEG, _dshl(v2[...], 1, _NEG))
    row_c1 = jnp.logical_and(valid, iou > fr)
    col_c1 = jnp.logical_and(valid, iou > fc)

    blk_p = _dshr(sp_f, 1, 1.0)
    blk_t = _dshr(st_f, 1, 1.0)
    v1[...] = iou
    f1[...] = blk_p
    v2[...] = iou
    f2[...] = blk_t
    _seg_bwd_loop(v1, f1, v2, f2)
    br = jnp.where(blk_p > 0.5, _NEG, _dshr(v1[...], 1, _NEG))
    bc = jnp.where(blk_t > 0.5, _NEG, _dshr(v2[...], 1, _NEG))

    is_row_best = jnp.logical_and(row_c1, iou >= br)
    is_col_best = jnp.logical_and(col_c1, iou >= bc)
    passes = iou >= _IOU_THRESHOLD
    mutual = jnp.logical_and(jnp.logical_and(is_row_best, is_col_best), passes)
    mf = jnp.where(mutual, 1.0, 0.0)

    v1[...] = mf
    f1[...] = st_f
    v2[...] = mf
    f2[...] = sp_f
    _seg_fwd_loop(v1, f1, v2, f2)
    ft = jnp.where(start_t, _NEG, _dshl(v1[...], 1, _NEG))
    fpp = jnp.where(start_p, _NEG, _dshl(v2[...], 1, _NEG))
    any_t = jnp.logical_or(mutual, ft > 0.0)
    any_p = jnp.logical_or(mutual, fpp > 0.0)

    v1[...] = mf
    f1[...] = blk_t
    v2[...] = mf
    f2[...] = blk_p
    _seg_bwd_loop(v1, f1, v2, f2)
    bt = jnp.where(blk_t > 0.5, _NEG, _dshr(v1[...], 1, _NEG))
    bpp = jnp.where(blk_p > 0.5, _NEG, _dshr(v2[...], 1, _NEG))
    any_mut_t = jnp.logical_or(any_t, bt > 0.0)
    any_mut_p = jnp.logical_or(any_p, bpp > 0.0)

    term1 = jnp.logical_and(jnp.logical_and(is_row_best, passes),
                            jnp.logical_not(any_mut_t))
    term2 = jnp.logical_and(jnp.logical_and(is_col_best, passes),
                            jnp.logical_not(any_mut_p))

    one_f = jnp.float32(1.0)
    zero_f = jnp.float32(0.0)
    as_f = lambda m: jnp.where(m, one_f, zero_f)

    tp_r = jnp.sum(as_f(mutual), axis=1, keepdims=True)
    one_r = jnp.sum(as_f(term1) + as_f(term2), axis=1, keepdims=True)
    a_r = jnp.sum(as_f(a_flag), axis=1, keepdims=True)
    b_r = jnp.sum(st_f, axis=1, keepdims=True)

    matched = tp_r + one_r
    fp_row = jnp.maximum(a_r - matched, 0.0)
    fn_row = jnp.maximum(b_r - matched, 0.0)

    tp_tot = acc_ref[0] + jnp.sum(tp_r)
    fp_tot = acc_ref[1] + jnp.sum(fp_row)
    fn_tot = acc_ref[2] + jnp.sum(fn_row)
    acc_ref[0] = tp_tot
    acc_ref[1] = fp_tot
    acc_ref[2] = fn_tot

    den_p = tp_tot + fp_tot
    prec = jnp.where(den_p > 0, tp_tot / jnp.where(den_p > 0, den_p, 1.0), 0.0)
    den_r = tp_tot + fn_tot
    rec = jnp.where(den_r > 0, tp_tot / jnp.where(den_r > 0, den_r, 1.0), 0.0)
    den_f = 2.0 * tp_tot + fp_tot + fn_tot
    f1s = jnp.where(tp_tot > 0,
                    (2.0 * tp_tot) / jnp.where(den_f > 0, den_f, 1.0), 0.0)

    lane = jax.lax.broadcasted_iota(jnp.int32, (1, 128), 1)
    vec = jnp.where(lane == 0, prec,
                    jnp.where(lane == 1, rec,
                              jnp.where(lane == 2, f1s, 0.0)))
    res_ref[...] = vec


def _run(output, target, interpret=False):
    grid = output.shape[0] // _ROWS
    big = pltpu.VMEM((_ROWS, _N), jnp.float32)
    res = pl.pallas_call(
        _body,
        grid=(grid,),
        in_specs=[
            pl.BlockSpec((_ROWS, _N), lambda i: (i, 0)),
            pl.BlockSpec((_ROWS, _N), lambda i: (i, 0)),
        ],
        out_specs=pl.BlockSpec((1, 128), lambda i: (0, 0)),
        out_shape=jax.ShapeDtypeStruct((1, 128), jnp.float32),
        scratch_shapes=[big, big, big, big, big, big, big, big, big,
                        pltpu.SMEM((3,), jnp.float32)],
        compiler_params=pltpu.CompilerParams(
            dimension_semantics=("arbitrary",),
        ),
        interpret=interpret,
    )(output, target)
    return res[0, :3]


def kernel(output, target):
    return _run(output, target)


# speedup vs baseline: 1135.7601x; 7.0541x over previous
import functools

import jax
import jax.numpy as jnp
from jax import lax
from jax.experimental import pallas as pl
from jax.experimental.pallas import tpu as pltpu
from jax.experimental.pallas import tpu_sc as plsc

_TH = 0.05
_TAU = 0.3
_LEN = 10

_N = 32768
_CHUNK = 2048
_NCHUNK = _N // _CHUNK
_NGRP = _CHUNK // 16
_MAXE = 16384
_MAXP = 3008


def _it16():
    return lax.broadcasted_iota(jnp.int32, (16,), 0)


def _sc_body(out_hbm, tgt_hbm, res_hbm, cb_f, cb_i, raw_s, raw_e, ps, pe,
             rmax, ridx, cmax, cidx, t_taken, p_mut, v16, acc):
    wid = lax.axis_index("s") * 2 + lax.axis_index("c")

    def extract(row, hbm, cb, is_pred):
        def test(v):
            return (v >= _TH) if is_pred else (v != 0)

        def chunk_body(ch, carry):
            cnt, prev = carry
            pltpu.sync_copy(hbm.at[row, pl.ds(ch * _CHUNK, _CHUNK)], cb)

            def grp_body(g, c2):
                cnt, prev = c2
                it = _it16()
                base = g * 16
                ones = jnp.ones((16,), jnp.int32)
                zeros = jnp.zeros((16,), jnp.int32)
                v = cb[pl.ds(base, 16)]
                m = test(v)
                pv_raw = plsc.load_gather(cb, [jnp.maximum(base - 1 + it, 0)])
                pv_b = test(pv_raw)
                first = jnp.logical_and(it == 0,
                                        jnp.full((16,), g, jnp.int32) == 0)
                pv_b = jnp.where(first, jnp.full((16,), prev, jnp.int32) == 1,
                                 pv_b)
                startm = jnp.logical_and(m, jnp.logical_not(pv_b))
                endm = jnp.logical_and(jnp.logical_not(m), pv_b)
                gpos = ch * _CHUNK + base + it
                cs = plsc.cumsum(jnp.where(startm, ones, zeros))
                plsc.store_scatter(raw_s, [cnt[0] + cs - 1], gpos, mask=startm)
                ce = plsc.cumsum(jnp.where(endm, ones, zeros))
                plsc.store_scatter(raw_e, [cnt[1] + ce - 1], gpos, mask=endm)
                cnt = (cnt[0] + jnp.max(cs), cnt[1] + jnp.max(ce))
                prev = jnp.max(jnp.where(jnp.logical_and(it == 15, m),
                                         ones, zeros))
                return cnt, prev

            return lax.fori_loop(0, _NGRP, grp_body, (cnt, prev))

        (cnt_s, cnt_e), prev = lax.fori_loop(
            0, _NCHUNK, chunk_body,
            ((jnp.int32(0), jnp.int32(0)), jnp.int32(0)))
        it = _it16()
        tail = jnp.logical_and(it == 0, jnp.full((16,), prev, jnp.int32) == 1)
        plsc.store_scatter(raw_e, [jnp.full((16,), cnt_e, jnp.int32)],
                           jnp.full((16,), _N, jnp.int32), mask=tail)
        return cnt_s

    def filter_preds(cnt):
        def body(g, a):
            it = _it16()
            i = g * 16 + it
            inb = i < cnt
            ic = jnp.minimum(i, jnp.maximum(cnt - 1, 0))
            s = plsc.load_gather(raw_s, [ic])
            e = plsc.load_gather(raw_e, [ic])
            ok = jnp.logical_and(inb, (e - s) >= _LEN)
            cs = plsc.cumsum(jnp.where(ok, jnp.ones((16,), jnp.int32),
                                       jnp.zeros((16,), jnp.int32)))
            idx = a + cs - 1
            plsc.store_scatter(ps, [idx], s, mask=ok)
            plsc.store_scatter(pe, [idx], e, mask=ok)
            return a + jnp.max(cs)

        return lax.fori_loop(0, (cnt + 15) // 16, body, jnp.int32(0))

    def merge(a_s, a_e, b_s, b_e, n_a, n_b, omax, oidx):
        it = _it16()
        per = (n_a + 15) // 16
        lo = it * per
        hi = jnp.minimum(lo + per, n_a)
        amax = jnp.maximum(n_a - 1, 0)
        bmax = jnp.maximum(n_b - 1, 0)
        ps0 = plsc.load_gather(a_s, [jnp.minimum(lo, amax)])

        def bs_body(_, c):
            lo_k, hi_k = c
            act = lo_k < hi_k
            mid = (lo_k + hi_k) >> 1
            tem = plsc.load_gather(b_e, [jnp.minimum(mid, bmax)])
            goright = jnp.logical_and(act, tem <= ps0)
            lo_k = jnp.where(goright, mid + 1, lo_k)
            hi_k = jnp.where(jnp.logical_and(act, jnp.logical_not(goright)),
                             mid, hi_k)
            return lo_k, hi_k

        k0, _ = lax.fori_loop(0, 14, bs_body,
                              (jnp.zeros((16,), jnp.int32),
                               jnp.full((16,), n_b, jnp.int32)))

        def cond(c):
            j, k, bv, bk = c
            return jnp.any(j < hi)

        def step(c):
            j, k, bv, bk = c
            act = j < hi
            jj = jnp.minimum(j, amax)
            kk = jnp.minimum(k, bmax)
            asj = plsc.load_gather(a_s, [jj])
            aej = plsc.load_gather(a_e, [jj])
            bsk = plsc.load_gather(b_s, [kk])
            bek = plsc.load_gather(b_e, [kk])
            kin = k < jnp.full((16,), n_b, jnp.int32)
            inter = jnp.minimum(aej, bek) - jnp.maximum(asj, bsk)
            ov = jnp.logical_and(jnp.logical_and(act, kin), inter > 0)
            la = (aej - asj).astype(jnp.float32)
            lb = (bek - bsk).astype(jnp.float32)
            inf_ = inter.astype(jnp.float32)
            den = jnp.where(ov, la + lb - inf_, jnp.ones((16,), jnp.float32))
            iou = inf_ / den
            better = jnp.logical_and(ov, iou > bv)
            bv = jnp.where(better, iou, bv)
            bk = jnp.where(better, kk, bk)
            adv_j = jnp.logical_and(
                act, jnp.logical_or(jnp.logical_not(kin), aej <= bek))
            adv_k = jnp.logical_and(jnp.logical_and(act, kin), bek <= aej)
            plsc.store_scatter(omax, [jj], bv, mask=adv_j)
            plsc.store_scatter(oidx, [jj], bk, mask=adv_j)
            j = jnp.where(adv_j, j + 1, j)
            bv = jnp.where(adv_j, jnp.full((16,), -1.0, jnp.float32), bv)
            bk = jnp.where(adv_j, jnp.zeros((16,), jnp.int32), bk)
            k = jnp.where(adv_k, k + 1, k)
            return j, k, bv, bk

        lax.while_loop(cond, step,
                       (lo, k0, jnp.full((16,), -1.0, jnp.float32),
                        jnp.zeros((16,), jnp.int32)))

    def match(n_a, n_b):
        it = _it16()
        amax = jnp.maximum(n_a - 1, 0)
        bmax = jnp.maximum(n_b - 1, 0)

        def zero_body(g, ref, n):
            def body(gg, _):
                i = gg * 16 + it
                plsc.store_scatter(ref, [jnp.minimum(i, n - 1)],
                                   jnp.zeros((16,), jnp.int32), mask=i < n)
                return 0
            return body

        lax.fori_loop(0, (n_b + 15) // 16, zero_body(0, t_taken, n_b), 0)
        lax.fori_loop(0, (n_a + 15) // 16, zero_body(0, p_mut, n_a), 0)

        def pass_a(g, tp):
            i = g * 16 + it
            inb = i < n_a
            ii = jnp.minimum(i, amax)
            rv = plsc.load_gather(rmax, [ii])
            rk = plsc.load_gather(ridx, [ii])
            cj = plsc.load_gather(cidx, [jnp.minimum(rk, bmax)])
            i2 = jnp.logical_and(jnp.logical_and(inb, rv >= _TAU), cj == i)
            plsc.store_scatter(p_mut, [ii], jnp.ones((16,), jnp.int32),
                               mask=i2)
            plsc.store_scatter(t_taken, [rk], jnp.ones((16,), jnp.int32),
                               mask=i2)
            return tp + jnp.sum(jnp.where(i2, jnp.ones((16,), jnp.int32),
                                          jnp.zeros((16,), jnp.int32)))

        tp = lax.fori_loop(0, (n_a + 15) // 16, pass_a, jnp.int32(0))

        def pass_b(g, one):
            i = g * 16 + it
            inb = i < n_a
            ii = jnp.minimum(i, amax)
            rv = plsc.load_gather(rmax, [ii])
            rk = plsc.load_gather(ridx, [ii])
            im = plsc.load_gather(p_mut, [ii])
            tk = plsc.load_gather(t_taken, [jnp.minimum(rk, bmax)])
            m2 = jnp.logical_and(jnp.logical_and(inb, rv >= _TAU), im == 0)
            keep = jnp.logical_and(m2, tk == 0)
            return one + jnp.sum(jnp.where(keep, jnp.ones((16,), jnp.int32),
                                           jnp.zeros((16,), jnp.int32)))

        one = lax.fori_loop(0, (n_a + 15) // 16, pass_b, jnp.int32(0))

        def pass_c(g, one):
            i = g * 16 + it
            inb = i < n_b
            ii = jnp.minimum(i, bmax)
            cv = plsc.load_gather(cmax, [ii])
            ck = plsc.load_gather(cidx, [ii])
            rj = plsc.load_gather(ridx, [jnp.minimum(ck, amax)])
            pm = plsc.load_gather(p_mut, [jnp.minimum(ck, amax)])
            i1 = jnp.logical_and(cv >= _TAU, rj == i)
            m1 = jnp.logical_and(jnp.logical_and(inb, cv >= _TAU),
                                 jnp.logical_not(i1))
            keep = jnp.logical_and(m1, pm == 0)
            return one + jnp.sum(jnp.where(keep, jnp.ones((16,), jnp.int32),
                                           jnp.zeros((16,), jnp.int32)))

        one = lax.fori_loop(0, (n_b + 15) // 16, pass_c, one)
        acc[0] = tp
        acc[1] = one

    tp_all = jnp.float32(0.0)
    fp_all = jnp.float32(0.0)
    fn_all = jnp.float32(0.0)
    for r in range(2):
        row = wid * 2 + r
        cnt_p = extract(row, out_hbm, cb_f, True)
        n_a = filter_preds(cnt_p)
        n_b = extract(row, tgt_hbm, cb_i, False)
        acc[0] = jnp.int32(0)
        acc[1] = jnp.int32(0)
        both = jnp.logical_and(n_a > 0, n_b > 0)

        @pl.when(both)
        def _do():
            merge(ps, pe, raw_s, raw_e, n_a, n_b, rmax, ridx)
            merge(raw_s, raw_e, ps, pe, n_b, n_a, cmax, cidx)
            match(n_a, n_b)

        tp = acc[0].astype(jnp.float32)
        one = acc[1].astype(jnp.float32)
        matched = tp + one
        tp_all = tp_all + tp
        fp_all = fp_all + jnp.maximum(n_a.astype(jnp.float32) - matched, 0.0)
        fn_all = fn_all + jnp.maximum(n_b.astype(jnp.float32) - matched, 0.0)

    it = _it16()
    v16[...] = jnp.where(it == 0, jnp.full((16,), tp_all, jnp.float32),
                         jnp.where(it == 1, jnp.full((16,), fp_all, jnp.float32),
                                   jnp.where(it == 2, jnp.full((16,), fn_all, jnp.float32),
                                             jnp.zeros((16,), jnp.float32))))
    pltpu.sync_copy(v16, res_hbm.at[wid, pl.ds(0, 16)])


def _make_sc():
    mesh = plsc.VectorSubcoreMesh(core_axis_name="c", subcore_axis_name="s")
    return functools.partial(
        pl.kernel,
        mesh=mesh,
        out_type=jax.ShapeDtypeStruct((32, 128), jnp.float32),
        compiler_params=pltpu.CompilerParams(needs_layout_passes=False),
        scratch_types=[
            pltpu.VMEM((_CHUNK,), jnp.float32),
            pltpu.VMEM((_CHUNK,), jnp.int32),
            pltpu.VMEM((_MAXE,), jnp.int32),
            pltpu.VMEM((_MAXE,), jnp.int32),
            pltpu.VMEM((_MAXP,), jnp.int32),
            pltpu.VMEM((_MAXP,), jnp.int32),
            pltpu.VMEM((_MAXP,), jnp.float32),
            pltpu.VMEM((_MAXP,), jnp.int32),
            pltpu.VMEM((_MAXE,), jnp.float32),
            pltpu.VMEM((_MAXE,), jnp.int32),
            pltpu.VMEM((_MAXE,), jnp.int32),
            pltpu.VMEM((_MAXP,), jnp.int32),
            pltpu.VMEM((16,), jnp.float32),
            pltpu.SMEM((4,), jnp.int32),
        ],
    )(_sc_body)


def _fin_body(parts_ref, res_ref):
    x = parts_ref[...]
    lane = lax.broadcasted_iota(jnp.int32, (32, 128), 1)
    zero = jnp.zeros((32, 128), jnp.float32)
    tp = jnp.sum(jnp.where(lane == 0, x, zero))
    fp = jnp.sum(jnp.where(lane == 1, x, zero))
    fn = jnp.sum(jnp.where(lane == 2, x, zero))
    den_p = tp + fp
    prec = jnp.where(den_p > 0, tp / jnp.where(den_p > 0, den_p, 1.0), 0.0)
    den_r = tp + fn
    rec = jnp.where(den_r > 0, tp / jnp.where(den_r > 0, den_r, 1.0), 0.0)
    den_f = 2.0 * tp + fp + fn
    f1 = jnp.where(tp > 0, (2.0 * tp) / jnp.where(den_f > 0, den_f, 1.0), 0.0)
    lo = lax.broadcasted_iota(jnp.int32, (1, 128), 1)
    res_ref[...] = jnp.where(lo == 0, prec,
                             jnp.where(lo == 1, rec,
                                       jnp.where(lo == 2, f1, 0.0)))


def kernel(output, target):
    parts = _make_sc()(output, target)
    res = pl.pallas_call(
        _fin_body,
        out_shape=jax.ShapeDtypeStruct((1, 128), jnp.float32),
    )(parts)
    return res[0, :3]


# speedup vs baseline: 1199.0969x; 1.0558x over previous
import functools

import jax
import jax.numpy as jnp
from jax import lax
from jax.experimental import pallas as pl
from jax.experimental.pallas import tpu as pltpu
from jax.experimental.pallas import tpu_sc as plsc

_TH = 0.05
_TAU = 0.3
_LEN = 10

_N = 32768
_CHUNK = 2048
_NCHUNK = _N // _CHUNK
_NGRP = _CHUNK // 16
_MAXE = 16384
_MAXP = 3008


def _it16():
    return lax.broadcasted_iota(jnp.int32, (16,), 0)


def _sc_body(out_hbm, tgt_hbm, res_hbm, cb_f, cb_i, raw_s, raw_e, ps, pe,
             rmax, ridx, cmax, cidx, t_taken, p_mut, v16, acc):
    wid = lax.axis_index("s") * 2 + lax.axis_index("c")

    def extract(row, hbm, cb, is_pred):
        def test(v):
            return (v >= _TH) if is_pred else (v != 0)

        ones = jnp.ones((16,), jnp.int32)
        zeros = jnp.zeros((16,), jnp.int32)
        it = _it16()

        def chunk_body(ch, carry):
            cnt_s, cnt_e, prevv = carry
            pltpu.sync_copy(hbm.at[row, pl.ds(ch * _CHUNK, _CHUNK)], cb)

            def grp_body(g, c2):
                cnt_s, cnt_e = c2
                for u in range(2):
                    base = (g * 2 + u) * 16
                    v = cb[pl.ds(base, 16)]
                    m = test(v)
                    pv_raw = plsc.load_gather(
                        cb, [jnp.maximum(base - 1 + it, 0)])
                    pv_b = test(pv_raw)
                    if u == 0:
                        first = jnp.logical_and(
                            it == 0, jnp.full((16,), g, jnp.int32) == 0)
                        pv_b = jnp.where(first, prevv == 1, pv_b)
                    startm = jnp.logical_and(m, jnp.logical_not(pv_b))
                    endm = jnp.logical_and(jnp.logical_not(m), pv_b)
                    gpos = ch * _CHUNK + base + it
                    cs = plsc.cumsum(jnp.where(startm, ones, zeros))
                    plsc.store_scatter(raw_s, [cnt_s + cs - 1], gpos,
                                       mask=startm)
                    ce = plsc.cumsum(jnp.where(endm, ones, zeros))
                    plsc.store_scatter(raw_e, [cnt_e + ce - 1], gpos,
                                       mask=endm)
                    cnt_s = cnt_s + plsc.all_reduce_population_count(startm)
                    cnt_e = cnt_e + plsc.all_reduce_population_count(endm)
                return cnt_s, cnt_e

            cnt_s, cnt_e = lax.fori_loop(0, _NGRP // 2, grp_body,
                                         (cnt_s, cnt_e))
            lastv = plsc.load_gather(cb, [jnp.full((16,), _CHUNK - 1,
                                                   jnp.int32)])
            prevv = jnp.where(test(lastv), ones, zeros)
            return cnt_s, cnt_e, prevv

        cnt_s, cnt_e, prevv = lax.fori_loop(
            0, _NCHUNK, chunk_body, (zeros, zeros, zeros))
        tail = jnp.logical_and(it == 0, prevv == 1)
        plsc.store_scatter(raw_e, [cnt_e],
                           jnp.full((16,), _N, jnp.int32), mask=tail)
        return jnp.max(cnt_s)

    def filter_preds(cnt):
        def body(g, a):
            it = _it16()
            i = g * 16 + it
            inb = i < cnt
            ic = jnp.minimum(i, jnp.maximum(cnt - 1, 0))
            s = plsc.load_gather(raw_s, [ic])
            e = plsc.load_gather(raw_e, [ic])
            ok = jnp.logical_and(inb, (e - s) >= _LEN)
            cs = plsc.cumsum(jnp.where(ok, jnp.ones((16,), jnp.int32),
                                       jnp.zeros((16,), jnp.int32)))
            idx = a + cs - 1
            plsc.store_scatter(ps, [idx], s, mask=ok)
            plsc.store_scatter(pe, [idx], e, mask=ok)
            return a + plsc.all_reduce_population_count(ok)

        a = lax.fori_loop(0, (cnt + 15) // 16, body,
                          jnp.zeros((16,), jnp.int32))
        return jnp.max(a)

    def merge(a_s, a_e, b_s, b_e, n_a, n_b, omax, oidx):
        it = _it16()
        per = (n_a + 15) // 16
        lo = it * per
        hi = jnp.minimum(lo + per, n_a)
        amax = jnp.maximum(n_a - 1, 0)
        bmax = jnp.maximum(n_b - 1, 0)
        ps0 = plsc.load_gather(a_s, [jnp.minimum(lo, amax)])

        def bs_body(_, c):
            lo_k, hi_k = c
            act = lo_k < hi_k
            mid = (lo_k + hi_k) >> 1
            tem = plsc.load_gather(b_e, [jnp.minimum(mid, bmax)])
            goright = jnp.logical_and(act, tem <= ps0)
            lo_k = jnp.where(goright, mid + 1, lo_k)
            hi_k = jnp.where(jnp.logical_and(act, jnp.logical_not(goright)),
                             mid, hi_k)
            return lo_k, hi_k

        k0, _ = lax.fori_loop(0, 14, bs_body,
                              (jnp.zeros((16,), jnp.int32),
                               jnp.full((16,), n_b, jnp.int32)))

        def cond(c):
            j, k, bv, bk = c
            return jnp.any(j < hi)

        def step(c):
            j, k, bv, bk = c
            act = j < hi
            jj = jnp.minimum(j, amax)
            kk = jnp.minimum(k, bmax)
            asj = plsc.load_gather(a_s, [jj])
            aej = plsc.load_gather(a_e, [jj])
            bsk = plsc.load_gather(b_s, [kk])
            bek = plsc.load_gather(b_e, [kk])
            kin = k < jnp.full((16,), n_b, jnp.int32)
            inter = jnp.minimum(aej, bek) - jnp.maximum(asj, bsk)
            ov = jnp.logical_and(jnp.logical_and(act, kin), inter > 0)
            la = (aej - asj).astype(jnp.float32)
            lb = (bek - bsk).astype(jnp.float32)
            inf_ = inter.astype(jnp.float32)
            den = jnp.where(ov, la + lb - inf_, jnp.ones((16,), jnp.float32))
            iou = inf_ / den
            better = jnp.logical_and(ov, iou > bv)
            bv = jnp.where(better, iou, bv)
            bk = jnp.where(better, kk, bk)
            adv_j = jnp.logical_and(
                act, jnp.logical_or(jnp.logical_not(kin), aej <= bek))
            adv_k = jnp.logical_and(jnp.logical_and(act, kin), bek <= aej)
            plsc.store_scatter(omax, [jj], bv, mask=adv_j)
            plsc.store_scatter(oidx, [jj], bk, mask=adv_j)
            j = jnp.where(adv_j, j + 1, j)
            bv = jnp.where(adv_j, jnp.full((16,), -1.0, jnp.float32), bv)
            bk = jnp.where(adv_j, jnp.zeros((16,), jnp.int32), bk)
            k = jnp.where(adv_k, k + 1, k)
            return j, k, bv, bk

        lax.while_loop(cond, step,
                       (lo, k0, jnp.full((16,), -1.0, jnp.float32),
                        jnp.zeros((16,), jnp.int32)))

    def match(n_a, n_b):
        it = _it16()
        amax = jnp.maximum(n_a - 1, 0)
        bmax = jnp.maximum(n_b - 1, 0)

        def zero_body(g, ref, n):
            def body(gg, _):
                i = gg * 16 + it
                plsc.store_scatter(ref, [jnp.minimum(i, n - 1)],
                                   jnp.zeros((16,), jnp.int32), mask=i < n)
                return 0
            return body

        lax.fori_loop(0, (n_b + 15) // 16, zero_body(0, t_taken, n_b), 0)
        lax.fori_loop(0, (n_a + 15) // 16, zero_body(0, p_mut, n_a), 0)

        def pass_a(g, tp):
            i = g * 16 + it
            inb = i < n_a
            ii = jnp.minimum(i, amax)
            rv = plsc.load_gather(rmax, [ii])
            rk = plsc.load_gather(ridx, [ii])
            cj = plsc.load_gather(cidx, [jnp.minimum(rk, bmax)])
            i2 = jnp.logical_and(jnp.logical_and(inb, rv >= _TAU), cj == i)
            plsc.store_scatter(p_mut, [ii], jnp.ones((16,), jnp.int32),
                               mask=i2)
            plsc.store_scatter(t_taken, [rk], jnp.ones((16,), jnp.int32),
                               mask=i2)
            return tp + jnp.sum(jnp.where(i2, jnp.ones((16,), jnp.int32),
                                          jnp.zeros((16,), jnp.int32)))

        tp = lax.fori_loop(0, (n_a + 15) // 16, pass_a, jnp.int32(0))

        def pass_b(g, one):
            i = g * 16 + it
            inb = i < n_a
            ii = jnp.minimum(i, amax)
            rv = plsc.load_gather(rmax, [ii])
            rk = plsc.load_gather(ridx, [ii])
            im = plsc.load_gather(p_mut, [ii])
            tk = plsc.load_gather(t_taken, [jnp.minimum(rk, bmax)])
            m2 = jnp.logical_and(jnp.logical_and(inb, rv >= _TAU), im == 0)
            keep = jnp.logical_and(m2, tk == 0)
            return one + jnp.sum(jnp.where(keep, jnp.ones((16,), jnp.int32),
                                           jnp.zeros((16,), jnp.int32)))

        one = lax.fori_loop(0, (n_a + 15) // 16, pass_b, jnp.int32(0))

        def pass_c(g, one):
            i = g * 16 + it
            inb = i < n_b
            ii = jnp.minimum(i, bmax)
            cv = plsc.load_gather(cmax, [ii])
            ck = plsc.load_gather(cidx, [ii])
            rj = plsc.load_gather(ridx, [jnp.minimum(ck, amax)])
            pm = plsc.load_gather(p_mut, [jnp.minimum(ck, amax)])
            i1 = jnp.logical_and(cv >= _TAU, rj == i)
            m1 = jnp.logical_and(jnp.logical_and(inb, cv >= _TAU),
                                 jnp.logical_not(i1))
            keep = jnp.logical_and(m1, pm == 0)
            return one + jnp.sum(jnp.where(keep, jnp.ones((16,), jnp.int32),
                                           jnp.zeros((16,), jnp.int32)))

        one = lax.fori_loop(0, (n_b + 15) // 16, pass_c, one)
        acc[0] = tp
        acc[1] = one

    tp_all = jnp.float32(0.0)
    fp_all = jnp.float32(0.0)
    fn_all = jnp.float32(0.0)
    for r in range(2):
        row = wid * 2 + r
        cnt_p = extract(row, out_hbm, cb_f, True)
        n_a = filter_preds(cnt_p)
        n_b = extract(row, tgt_hbm, cb_i, False)
        acc[0] = jnp.int32(0)
        acc[1] = jnp.int32(0)
        both = jnp.logical_and(n_a > 0, n_b > 0)

        @pl.when(both)
        def _do():
            merge(ps, pe, raw_s, raw_e, n_a, n_b, rmax, ridx)
            merge(raw_s, raw_e, ps, pe, n_b, n_a, cmax, cidx)
            match(n_a, n_b)

        tp = acc[0].astype(jnp.float32)
        one = acc[1].astype(jnp.float32)
        matched = tp + one
        tp_all = tp_all + tp
        fp_all = fp_all + jnp.maximum(n_a.astype(jnp.float32) - matched, 0.0)
        fn_all = fn_all + jnp.maximum(n_b.astype(jnp.float32) - matched, 0.0)

    it = _it16()
    v16[...] = jnp.where(it == 0, jnp.full((16,), tp_all, jnp.float32),
                         jnp.where(it == 1, jnp.full((16,), fp_all, jnp.float32),
                                   jnp.where(it == 2, jnp.full((16,), fn_all, jnp.float32),
                                             jnp.zeros((16,), jnp.float32))))
    pltpu.sync_copy(v16, res_hbm.at[wid, pl.ds(0, 16)])


def _make_sc():
    mesh = plsc.VectorSubcoreMesh(core_axis_name="c", subcore_axis_name="s")
    return functools.partial(
        pl.kernel,
        mesh=mesh,
        out_type=jax.ShapeDtypeStruct((32, 128), jnp.float32),
        compiler_params=pltpu.CompilerParams(needs_layout_passes=False),
        scratch_types=[
            pltpu.VMEM((_CHUNK,), jnp.float32),
            pltpu.VMEM((_CHUNK,), jnp.int32),
            pltpu.VMEM((_MAXE,), jnp.int32),
            pltpu.VMEM((_MAXE,), jnp.int32),
            pltpu.VMEM((_MAXP,), jnp.int32),
            pltpu.VMEM((_MAXP,), jnp.int32),
            pltpu.VMEM((_MAXP,), jnp.float32),
            pltpu.VMEM((_MAXP,), jnp.int32),
            pltpu.VMEM((_MAXE,), jnp.float32),
            pltpu.VMEM((_MAXE,), jnp.int32),
            pltpu.VMEM((_MAXE,), jnp.int32),
            pltpu.VMEM((_MAXP,), jnp.int32),
            pltpu.VMEM((16,), jnp.float32),
            pltpu.SMEM((4,), jnp.int32),
        ],
    )(_sc_body)


def _fin_body(parts_ref, res_ref):
    x = parts_ref[...]
    lane = lax.broadcasted_iota(jnp.int32, (32, 128), 1)
    zero = jnp.zeros((32, 128), jnp.float32)
    tp = jnp.sum(jnp.where(lane == 0, x, zero))
    fp = jnp.sum(jnp.where(lane == 1, x, zero))
    fn = jnp.sum(jnp.where(lane == 2, x, zero))
    den_p = tp + fp
    prec = jnp.where(den_p > 0, tp / jnp.where(den_p > 0, den_p, 1.0), 0.0)
    den_r = tp + fn
    rec = jnp.where(den_r > 0, tp / jnp.where(den_r > 0, den_r, 1.0), 0.0)
    den_f = 2.0 * tp + fp + fn
    f1 = jnp.where(tp > 0, (2.0 * tp) / jnp.where(den_f > 0, den_f, 1.0), 0.0)
    lo = lax.broadcasted_iota(jnp.int32, (1, 128), 1)
    res_ref[...] = jnp.where(lo == 0, prec,
                             jnp.where(lo == 1, rec,
                                       jnp.where(lo == 2, f1, 0.0)))


def kernel(output, target):
    parts = _make_sc()(output, target)
    res = pl.pallas_call(
        _fin_body,
        out_shape=jax.ShapeDtypeStruct((1, 128), jnp.float32),
    )(parts)
    return res[0, :3]


# speedup vs baseline: 1319.4706x; 1.1004x over previous
import functools

import jax
import jax.numpy as jnp
from jax import lax
from jax.experimental import pallas as pl
from jax.experimental.pallas import tpu as pltpu
from jax.experimental.pallas import tpu_sc as plsc

_TH = 0.05
_TAU = 0.3
_LEN = 10

_N = 32768
_CHUNK = 8192
_NCHUNK = _N // _CHUNK
_NGRP = _CHUNK // 16
_MAXE = 16384
_MAXP = 3008


def _it16():
    return lax.broadcasted_iota(jnp.int32, (16,), 0)


def _sc_body(out_hbm, tgt_hbm, res_hbm, cb_f, cb_i, raw_s, raw_e, ps, pe,
             rmax, ridx, cmax, cidx, t_taken, p_mut, v16, acc):
    wid = lax.axis_index("s") * 2 + lax.axis_index("c")

    def extract(row, hbm, cb, is_pred):
        def test(v):
            return (v >= _TH) if is_pred else (v != 0)

        ones = jnp.ones((16,), jnp.int32)
        zeros = jnp.zeros((16,), jnp.int32)
        it = _it16()

        def chunk_body(ch, carry):
            cnt_s, cnt_e, prevv = carry
            pltpu.sync_copy(hbm.at[row, pl.ds(ch * _CHUNK, _CHUNK)], cb)

            def grp_body(g, c2):
                cnt_s, cnt_e = c2
                for u in range(2):
                    base = (g * 2 + u) * 16
                    v = cb[pl.ds(base, 16)]
                    m = test(v)
                    pv_raw = plsc.load_gather(
                        cb, [jnp.maximum(base - 1 + it, 0)])
                    pv_b = test(pv_raw)
                    if u == 0:
                        first = jnp.logical_and(
                            it == 0, jnp.full((16,), g, jnp.int32) == 0)
                        pv_b = jnp.where(first, prevv == 1, pv_b)
                    startm = jnp.logical_and(m, jnp.logical_not(pv_b))
                    endm = jnp.logical_and(jnp.logical_not(m), pv_b)
                    gpos = ch * _CHUNK + base + it
                    cs = plsc.cumsum(jnp.where(startm, ones, zeros))
                    plsc.store_scatter(raw_s, [cnt_s + cs - 1], gpos,
                                       mask=startm)
                    ce = plsc.cumsum(jnp.where(endm, ones, zeros))
                    plsc.store_scatter(raw_e, [cnt_e + ce - 1], gpos,
                                       mask=endm)
                    cnt_s = cnt_s + plsc.all_reduce_population_count(startm)
                    cnt_e = cnt_e + plsc.all_reduce_population_count(endm)
                return cnt_s, cnt_e

            cnt_s, cnt_e = lax.fori_loop(0, _NGRP // 2, grp_body,
                                         (cnt_s, cnt_e))
            lastv = plsc.load_gather(cb, [jnp.full((16,), _CHUNK - 1,
                                                   jnp.int32)])
            prevv = jnp.where(test(lastv), ones, zeros)
            return cnt_s, cnt_e, prevv

        cnt_s, cnt_e, prevv = lax.fori_loop(
            0, _NCHUNK, chunk_body, (zeros, zeros, zeros))
        tail = jnp.logical_and(it == 0, prevv == 1)
        plsc.store_scatter(raw_e, [cnt_e],
                           jnp.full((16,), _N, jnp.int32), mask=tail)
        return jnp.max(cnt_s)

    def filter_preds(cnt):
        def body(g, a):
            it = _it16()
            i = g * 16 + it
            inb = i < cnt
            ic = jnp.minimum(i, jnp.maximum(cnt - 1, 0))
            s = plsc.load_gather(raw_s, [ic])
            e = plsc.load_gather(raw_e, [ic])
            ok = jnp.logical_and(inb, (e - s) >= _LEN)
            cs = plsc.cumsum(jnp.where(ok, jnp.ones((16,), jnp.int32),
                                       jnp.zeros((16,), jnp.int32)))
            idx = a + cs - 1
            plsc.store_scatter(ps, [idx], s, mask=ok)
            plsc.store_scatter(pe, [idx], e, mask=ok)
            return a + plsc.all_reduce_population_count(ok)

        a = lax.fori_loop(0, (cnt + 15) // 16, body,
                          jnp.zeros((16,), jnp.int32))
        return jnp.max(a)

    def merge(a_s, a_e, b_s, b_e, n_a, n_b, omax, oidx):
        it = _it16()
        per = (n_a + 15) // 16
        lo = it * per
        hi = jnp.minimum(lo + per, n_a)
        amax = jnp.maximum(n_a - 1, 0)
        bmax = jnp.maximum(n_b - 1, 0)
        ps0 = plsc.load_gather(a_s, [jnp.minimum(lo, amax)])

        def bs_body(_, c):
            lo_k, hi_k = c
            act = lo_k < hi_k
            mid = (lo_k + hi_k) >> 1
            tem = plsc.load_gather(b_e, [jnp.minimum(mid, bmax)])
            goright = jnp.logical_and(act, tem <= ps0)
            lo_k = jnp.where(goright, mid + 1, lo_k)
            hi_k = jnp.where(jnp.logical_and(act, jnp.logical_not(goright)),
                             mid, hi_k)
            return lo_k, hi_k

        k0, _ = lax.fori_loop(0, 14, bs_body,
                              (jnp.zeros((16,), jnp.int32),
                               jnp.full((16,), n_b, jnp.int32)))

        def cond(c):
            j, k, bv, bk = c
            return jnp.any(j < hi)

        def step(c):
            j, k, bv, bk = c
            act = j < hi
            jj = jnp.minimum(j, amax)
            kk = jnp.minimum(k, bmax)
            asj = plsc.load_gather(a_s, [jj])
            aej = plsc.load_gather(a_e, [jj])
            bsk = plsc.load_gather(b_s, [kk])
            bek = plsc.load_gather(b_e, [kk])
            kin = k < jnp.full((16,), n_b, jnp.int32)
            inter = jnp.minimum(aej, bek) - jnp.maximum(asj, bsk)
            ov = jnp.logical_and(jnp.logical_and(act, kin), inter > 0)
            la = (aej - asj).astype(jnp.float32)
            lb = (bek - bsk).astype(jnp.float32)
            inf_ = inter.astype(jnp.float32)
            den = jnp.where(ov, la + lb - inf_, jnp.ones((16,), jnp.float32))
            iou = inf_ / den
            better = jnp.logical_and(ov, iou > bv)
            bv = jnp.where(better, iou, bv)
            bk = jnp.where(better, kk, bk)
            adv_j = jnp.logical_and(
                act, jnp.logical_or(jnp.logical_not(kin), aej <= bek))
            adv_k = jnp.logical_and(jnp.logical_and(act, kin), bek <= aej)
            plsc.store_scatter(omax, [jj], bv, mask=adv_j)
            plsc.store_scatter(oidx, [jj], bk, mask=adv_j)
            j = jnp.where(adv_j, j + 1, j)
            bv = jnp.where(adv_j, jnp.full((16,), -1.0, jnp.float32), bv)
            bk = jnp.where(adv_j, jnp.zeros((16,), jnp.int32), bk)
            k = jnp.where(adv_k, k + 1, k)
            return j, k, bv, bk

        lax.while_loop(cond, step,
                       (lo, k0, jnp.full((16,), -1.0, jnp.float32),
                        jnp.zeros((16,), jnp.int32)))

    def match(n_a, n_b):
        it = _it16()
        amax = jnp.maximum(n_a - 1, 0)
        bmax = jnp.maximum(n_b - 1, 0)

        def zero_body(g, ref, n):
            def body(gg, _):
                i = gg * 16 + it
                plsc.store_scatter(ref, [jnp.minimum(i, n - 1)],
                                   jnp.zeros((16,), jnp.int32), mask=i < n)
                return 0
            return body

        lax.fori_loop(0, (n_b + 15) // 16, zero_body(0, t_taken, n_b), 0)
        lax.fori_loop(0, (n_a + 15) // 16, zero_body(0, p_mut, n_a), 0)

        def pass_a(g, tp):
            i = g * 16 + it
            inb = i < n_a
            ii = jnp.minimum(i, amax)
            rv = plsc.load_gather(rmax, [ii])
            rk = plsc.load_gather(ridx, [ii])
            cj = plsc.load_gather(cidx, [jnp.minimum(rk, bmax)])
            i2 = jnp.logical_and(jnp.logical_and(inb, rv >= _TAU), cj == i)
            plsc.store_scatter(p_mut, [ii], jnp.ones((16,), jnp.int32),
                               mask=i2)
            plsc.store_scatter(t_taken, [rk], jnp.ones((16,), jnp.int32),
                               mask=i2)
            return tp + jnp.sum(jnp.where(i2, jnp.ones((16,), jnp.int32),
                                          jnp.zeros((16,), jnp.int32)))

        tp = lax.fori_loop(0, (n_a + 15) // 16, pass_a, jnp.int32(0))

        def pass_b(g, one):
            i = g * 16 + it
            inb = i < n_a
            ii = jnp.minimum(i, amax)
            rv = plsc.load_gather(rmax, [ii])
            rk = plsc.load_gather(ridx, [ii])
            im = plsc.load_gather(p_mut, [ii])
            tk = plsc.load_gather(t_taken, [jnp.minimum(rk, bmax)])
            m2 = jnp.logical_and(jnp.logical_and(inb, rv >= _TAU), im == 0)
            keep = jnp.logical_and(m2, tk == 0)
            return one + jnp.sum(jnp.where(keep, jnp.ones((16,), jnp.int32),
                                           jnp.zeros((16,), jnp.int32)))

        one = lax.fori_loop(0, (n_a + 15) // 16, pass_b, jnp.int32(0))

        def pass_c(g, one):
            i = g * 16 + it
            inb = i < n_b
            ii = jnp.minimum(i, bmax)
            cv = plsc.load_gather(cmax, [ii])
            ck = plsc.load_gather(cidx, [ii])
            rj = plsc.load_gather(ridx, [jnp.minimum(ck, amax)])
            pm = plsc.load_gather(p_mut, [jnp.minimum(ck, amax)])
            i1 = jnp.logical_and(cv >= _TAU, rj == i)
            m1 = jnp.logical_and(jnp.logical_and(inb, cv >= _TAU),
                                 jnp.logical_not(i1))
            keep = jnp.logical_and(m1, pm == 0)
            return one + jnp.sum(jnp.where(keep, jnp.ones((16,), jnp.int32),
                                           jnp.zeros((16,), jnp.int32)))

        one = lax.fori_loop(0, (n_b + 15) // 16, pass_c, one)
        acc[0] = tp
        acc[1] = one

    tp_all = jnp.float32(0.0)
    fp_all = jnp.float32(0.0)
    fn_all = jnp.float32(0.0)
    for r in range(2):
        row = wid * 2 + r
        cnt_p = extract(row, out_hbm, cb_f, True)
        n_a = filter_preds(cnt_p)
        n_b = extract(row, tgt_hbm, cb_i, False)
        acc[0] = jnp.int32(0)
        acc[1] = jnp.int32(0)
        both = jnp.logical_and(n_a > 0, n_b > 0)

        @pl.when(both)
        def _do():
            merge(ps, pe, raw_s, raw_e, n_a, n_b, rmax, ridx)
            merge(raw_s, raw_e, ps, pe, n_b, n_a, cmax, cidx)
            match(n_a, n_b)

        tp = acc[0].astype(jnp.float32)
        one = acc[1].astype(jnp.float32)
        matched = tp + one
        tp_all = tp_all + tp
        fp_all = fp_all + jnp.maximum(n_a.astype(jnp.float32) - matched, 0.0)
        fn_all = fn_all + jnp.maximum(n_b.astype(jnp.float32) - matched, 0.0)

    it = _it16()
    v16[...] = jnp.where(it == 0, jnp.full((16,), tp_all, jnp.float32),
                         jnp.where(it == 1, jnp.full((16,), fp_all, jnp.float32),
                                   jnp.where(it == 2, jnp.full((16,), fn_all, jnp.float32),
                                             jnp.zeros((16,), jnp.float32))))
    pltpu.sync_copy(v16, res_hbm.at[wid, pl.ds(0, 16)])


def _make_sc():
    mesh = plsc.VectorSubcoreMesh(core_axis_name="c", subcore_axis_name="s")
    return functools.partial(
        pl.kernel,
        mesh=mesh,
        out_type=jax.ShapeDtypeStruct((32, 128), jnp.float32),
        compiler_params=pltpu.CompilerParams(needs_layout_passes=False),
        scratch_types=[
            pltpu.VMEM((_CHUNK,), jnp.float32),
            pltpu.VMEM((_CHUNK,), jnp.int32),
            pltpu.VMEM((_MAXE,), jnp.int32),
            pltpu.VMEM((_MAXE,), jnp.int32),
            pltpu.VMEM((_MAXP,), jnp.int32),
            pltpu.VMEM((_MAXP,), jnp.int32),
            pltpu.VMEM((_MAXP,), jnp.float32),
            pltpu.VMEM((_MAXP,), jnp.int32),
            pltpu.VMEM((_MAXE,), jnp.float32),
            pltpu.VMEM((_MAXE,), jnp.int32),
            pltpu.VMEM((_MAXE,), jnp.int32),
            pltpu.VMEM((_MAXP,), jnp.int32),
            pltpu.VMEM((16,), jnp.float32),
            pltpu.SMEM((4,), jnp.int32),
        ],
    )(_sc_body)


def _fin_body(parts_ref, res_ref):
    x = parts_ref[...]
    lane = lax.broadcasted_iota(jnp.int32, (32, 128), 1)
    zero = jnp.zeros((32, 128), jnp.float32)
    tp = jnp.sum(jnp.where(lane == 0, x, zero))
    fp = jnp.sum(jnp.where(lane == 1, x, zero))
    fn = jnp.sum(jnp.where(lane == 2, x, zero))
    den_p = tp + fp
    prec = jnp.where(den_p > 0, tp / jnp.where(den_p > 0, den_p, 1.0), 0.0)
    den_r = tp + fn
    rec = jnp.where(den_r > 0, tp / jnp.where(den_r > 0, den_r, 1.0), 0.0)
    den_f = 2.0 * tp + fp + fn
    f1 = jnp.where(tp > 0, (2.0 * tp) / jnp.where(den_f > 0, den_f, 1.0), 0.0)
    lo = lax.broadcasted_iota(jnp.int32, (1, 128), 1)
    res_ref[...] = jnp.where(lo == 0, prec,
                             jnp.where(lo == 1, rec,
                                       jnp.where(lo == 2, f1, 0.0)))


def kernel(output, target):
    parts = _make_sc()(output, target)
    res = pl.pallas_call(
        _fin_body,
        out_shape=jax.ShapeDtypeStruct((1, 128), jnp.float32),
    )(parts)
    return res[0, :3]


# speedup vs baseline: 1350.0150x; 1.0231x over previous
import functools

import jax
import jax.numpy as jnp
from jax import lax
from jax.experimental import pallas as pl
from jax.experimental.pallas import tpu as pltpu
from jax.experimental.pallas import tpu_sc as plsc

_TH = 0.05
_TAU = 0.3
_LEN = 10

_N = 32768
_CHUNK = 8192
_NCHUNK = _N // _CHUNK
_NGRP = _CHUNK // 16
_MAXE = 16384
_MAXP = 3008


def _it16():
    return lax.broadcasted_iota(jnp.int32, (16,), 0)


def _sc_body(out_hbm, tgt_hbm, res_hbm, cb_f, cb_i, raw_b, ps, pe,
             rmax, ridx, cmax, cidx, t_taken, p_mut, v16, acc):
    wid = lax.axis_index("s") * 2 + lax.axis_index("c")

    def extract(row, hbm, cb, is_pred):
        def test(v):
            return (v >= _TH) if is_pred else (v != 0)

        ones = jnp.ones((16,), jnp.int32)
        zeros = jnp.zeros((16,), jnp.int32)
        it = _it16()

        def chunk_body(ch, carry):
            cnt_b, prevv = carry
            pltpu.sync_copy(hbm.at[row, pl.ds(ch * _CHUNK, _CHUNK)], cb)

            def grp_body(g, cnt_b):
                for u in range(4):
                    base = (g * 4 + u) * 16
                    v = cb[pl.ds(base, 16)]
                    m = test(v)
                    pv_raw = plsc.load_gather(
                        cb, [jnp.maximum(base - 1 + it, 0)])
                    pv_b = test(pv_raw)
                    if u == 0:
                        first = jnp.logical_and(
                            it == 0, jnp.full((16,), g, jnp.int32) == 0)
                        pv_b = jnp.where(first, prevv == 1, pv_b)
                    bm = m != pv_b
                    gpos = ch * _CHUNK + base + it
                    cs = plsc.cumsum(jnp.where(bm, ones, zeros))
                    plsc.store_scatter(raw_b, [cnt_b + cs - 1], gpos,
                                       mask=bm)
                    cnt_b = cnt_b + plsc.all_reduce_population_count(bm)
                return cnt_b

            cnt_b = lax.fori_loop(0, _NGRP // 4, grp_body, cnt_b)
            lastv = plsc.load_gather(cb, [jnp.full((16,), _CHUNK - 1,
                                                   jnp.int32)])
            prevv = jnp.where(test(lastv), ones, zeros)
            return cnt_b, prevv

        cnt_b, prevv = lax.fori_loop(
            0, _NCHUNK, chunk_body, (zeros, zeros))
        tail = jnp.logical_and(it == 0, prevv == 1)
        plsc.store_scatter(raw_b, [cnt_b],
                           jnp.full((16,), _N, jnp.int32), mask=tail)
        cnt_b = cnt_b + jnp.where(prevv == 1, ones, zeros)
        return jnp.max(cnt_b) >> 1

    def filter_preds(cnt):
        def body(g, a):
            it = _it16()
            i = g * 16 + it
            inb = i < cnt
            ic = jnp.minimum(i, jnp.maximum(cnt - 1, 0))
            s = plsc.load_gather(raw_b, [2 * ic])
            e = plsc.load_gather(raw_b, [2 * ic + 1])
            ok = jnp.logical_and(inb, (e - s) >= _LEN)
            cs = plsc.cumsum(jnp.where(ok, jnp.ones((16,), jnp.int32),
                                       jnp.zeros((16,), jnp.int32)))
            idx = a + cs - 1
            plsc.store_scatter(ps, [idx], s, mask=ok)
            plsc.store_scatter(pe, [idx], e, mask=ok)
            return a + plsc.all_reduce_population_count(ok)

        a = lax.fori_loop(0, (cnt + 15) // 16, body,
                          jnp.zeros((16,), jnp.int32))
        return jnp.max(a)

    def merge(get_a, get_b, n_a, n_b, omax, oidx):
        it = _it16()
        per = (n_a + 15) // 16
        lo = it * per
        hi = jnp.minimum(lo + per, n_a)
        amax = jnp.maximum(n_a - 1, 0)
        bmax = jnp.maximum(n_b - 1, 0)
        ps0, _ = get_a(jnp.minimum(lo, amax))

        def bs_body(_, c):
            lo_k, hi_k = c
            act = lo_k < hi_k
            mid = (lo_k + hi_k) >> 1
            _, tem = get_b(jnp.minimum(mid, bmax))
            goright = jnp.logical_and(act, tem <= ps0)
            lo_k = jnp.where(goright, mid + 1, lo_k)
            hi_k = jnp.where(jnp.logical_and(act, jnp.logical_not(goright)),
                             mid, hi_k)
            return lo_k, hi_k

        k0, _ = lax.fori_loop(0, 14, bs_body,
                              (jnp.zeros((16,), jnp.int32),
                               jnp.full((16,), n_b, jnp.int32)))

        def cond(c):
            j, k, bv, bk = c
            return jnp.any(j < hi)

        def step(c):
            j, k, bv, bk = c
            act = j < hi
            jj = jnp.minimum(j, amax)
            kk = jnp.minimum(k, bmax)
            asj, aej = get_a(jj)
            bsk, bek = get_b(kk)
            kin = k < jnp.full((16,), n_b, jnp.int32)
            inter = jnp.minimum(aej, bek) - jnp.maximum(asj, bsk)
            ov = jnp.logical_and(jnp.logical_and(act, kin), inter > 0)
            la = (aej - asj).astype(jnp.float32)
            lb = (bek - bsk).astype(jnp.float32)
            inf_ = inter.astype(jnp.float32)
            den = jnp.where(ov, la + lb - inf_, jnp.ones((16,), jnp.float32))
            iou = inf_ / den
            better = jnp.logical_and(ov, iou > bv)
            bv = jnp.where(better, iou, bv)
            bk = jnp.where(better, kk, bk)
            adv_j = jnp.logical_and(
                act, jnp.logical_or(jnp.logical_not(kin), aej <= bek))
            adv_k = jnp.logical_and(jnp.logical_and(act, kin), bek <= aej)
            plsc.store_scatter(omax, [jj], bv, mask=adv_j)
            plsc.store_scatter(oidx, [jj], bk, mask=adv_j)
            j = jnp.where(adv_j, j + 1, j)
            bv = jnp.where(adv_j, jnp.full((16,), -1.0, jnp.float32), bv)
            bk = jnp.where(adv_j, jnp.zeros((16,), jnp.int32), bk)
            k = jnp.where(adv_k, k + 1, k)
            return j, k, bv, bk

        lax.while_loop(cond, step,
                       (lo, k0, jnp.full((16,), -1.0, jnp.float32),
                        jnp.zeros((16,), jnp.int32)))

    def match(n_a, n_b):
        it = _it16()
        amax = jnp.maximum(n_a - 1, 0)
        bmax = jnp.maximum(n_b - 1, 0)

        def zero_body(g, ref, n):
            def body(gg, _):
                i = gg * 16 + it
                plsc.store_scatter(ref, [jnp.minimum(i, n - 1)],
                                   jnp.zeros((16,), jnp.int32), mask=i < n)
                return 0
            return body

        lax.fori_loop(0, (n_b + 15) // 16, zero_body(0, t_taken, n_b), 0)
        lax.fori_loop(0, (n_a + 15) // 16, zero_body(0, p_mut, n_a), 0)

        def pass_a(g, tp):
            i = g * 16 + it
            inb = i < n_a
            ii = jnp.minimum(i, amax)
            rv = plsc.load_gather(rmax, [ii])
            rk = plsc.load_gather(ridx, [ii])
            cj = plsc.load_gather(cidx, [jnp.minimum(rk, bmax)])
            i2 = jnp.logical_and(jnp.logical_and(inb, rv >= _TAU), cj == i)
            plsc.store_scatter(p_mut, [ii], jnp.ones((16,), jnp.int32),
                               mask=i2)
            plsc.store_scatter(t_taken, [rk], jnp.ones((16,), jnp.int32),
                               mask=i2)
            return tp + jnp.sum(jnp.where(i2, jnp.ones((16,), jnp.int32),
                                          jnp.zeros((16,), jnp.int32)))

        tp = lax.fori_loop(0, (n_a + 15) // 16, pass_a, jnp.int32(0))

        def pass_b(g, one):
            i = g * 16 + it
            inb = i < n_a
            ii = jnp.minimum(i, amax)
            rv = plsc.load_gather(rmax, [ii])
            rk = plsc.load_gather(ridx, [ii])
            im = plsc.load_gather(p_mut, [ii])
            tk = plsc.load_gather(t_taken, [jnp.minimum(rk, bmax)])
            m2 = jnp.logical_and(jnp.logical_and(inb, rv >= _TAU), im == 0)
            keep = jnp.logical_and(m2, tk == 0)
            return one + jnp.sum(jnp.where(keep, jnp.ones((16,), jnp.int32),
                                           jnp.zeros((16,), jnp.int32)))

        one = lax.fori_loop(0, (n_a + 15) // 16, pass_b, jnp.int32(0))

        def pass_c(g, one):
            i = g * 16 + it
            inb = i < n_b
            ii = jnp.minimum(i, bmax)
            cv = plsc.load_gather(cmax, [ii])
            ck = plsc.load_gather(cidx, [ii])
            rj = plsc.load_gather(ridx, [jnp.minimum(ck, amax)])
            pm = plsc.load_gather(p_mut, [jnp.minimum(ck, amax)])
            i1 = jnp.logical_and(cv >= _TAU, rj == i)
            m1 = jnp.logical_and(jnp.logical_and(inb, cv >= _TAU),
                                 jnp.logical_not(i1))
            keep = jnp.logical_and(m1, pm == 0)
            return one + jnp.sum(jnp.where(keep, jnp.ones((16,), jnp.int32),
                                           jnp.zeros((16,), jnp.int32)))

        one = lax.fori_loop(0, (n_b + 15) // 16, pass_c, one)
        acc[0] = tp
        acc[1] = one

    tp_all = jnp.float32(0.0)
    fp_all = jnp.float32(0.0)
    fn_all = jnp.float32(0.0)
    for r in range(2):
        row = wid * 2 + r
        cnt_p = extract(row, out_hbm, cb_f, True)
        n_a = filter_preds(cnt_p)
        n_b = extract(row, tgt_hbm, cb_i, False)
        acc[0] = jnp.int32(0)
        acc[1] = jnp.int32(0)
        both = jnp.logical_and(n_a > 0, n_b > 0)

        get_p = lambda i: (plsc.load_gather(ps, [i]),
                           plsc.load_gather(pe, [i]))
        get_t = lambda i: (plsc.load_gather(raw_b, [2 * i]),
                           plsc.load_gather(raw_b, [2 * i + 1]))

        @pl.when(both)
        def _do():
            merge(get_p, get_t, n_a, n_b, rmax, ridx)
            merge(get_t, get_p, n_b, n_a, cmax, cidx)
            match(n_a, n_b)

        tp = acc[0].astype(jnp.float32)
        one = acc[1].astype(jnp.float32)
        matched = tp + one
        tp_all = tp_all + tp
        fp_all = fp_all + jnp.maximum(n_a.astype(jnp.float32) - matched, 0.0)
        fn_all = fn_all + jnp.maximum(n_b.astype(jnp.float32) - matched, 0.0)

    it = _it16()
    v16[...] = jnp.where(it == 0, jnp.full((16,), tp_all, jnp.float32),
                         jnp.where(it == 1, jnp.full((16,), fp_all, jnp.float32),
                                   jnp.where(it == 2, jnp.full((16,), fn_all, jnp.float32),
                                             jnp.zeros((16,), jnp.float32))))
    pltpu.sync_copy(v16, res_hbm.at[wid, pl.ds(0, 16)])


def _make_sc():
    mesh = plsc.VectorSubcoreMesh(core_axis_name="c", subcore_axis_name="s")
    return functools.partial(
        pl.kernel,
        mesh=mesh,
        out_type=jax.ShapeDtypeStruct((32, 128), jnp.float32),
        compiler_params=pltpu.CompilerParams(needs_layout_passes=False),
        scratch_types=[
            pltpu.VMEM((_CHUNK,), jnp.float32),
            pltpu.VMEM((_CHUNK,), jnp.int32),
            pltpu.VMEM((2 * _MAXE,), jnp.int32),
            pltpu.VMEM((_MAXP,), jnp.int32),
            pltpu.VMEM((_MAXP,), jnp.int32),
            pltpu.VMEM((_MAXP,), jnp.float32),
            pltpu.VMEM((_MAXP,), jnp.int32),
            pltpu.VMEM((_MAXE,), jnp.float32),
            pltpu.VMEM((_MAXE,), jnp.int32),
            pltpu.VMEM((_MAXE,), jnp.int32),
            pltpu.VMEM((_MAXP,), jnp.int32),
            pltpu.VMEM((16,), jnp.float32),
            pltpu.SMEM((4,), jnp.int32),
        ],
    )(_sc_body)


def _fin_body(parts_ref, res_ref):
    x = parts_ref[...]
    lane = lax.broadcasted_iota(jnp.int32, (32, 128), 1)
    zero = jnp.zeros((32, 128), jnp.float32)
    tp = jnp.sum(jnp.where(lane == 0, x, zero))
    fp = jnp.sum(jnp.where(lane == 1, x, zero))
    fn = jnp.sum(jnp.where(lane == 2, x, zero))
    den_p = tp + fp
    prec = jnp.where(den_p > 0, tp / jnp.where(den_p > 0, den_p, 1.0), 0.0)
    den_r = tp + fn
    rec = jnp.where(den_r > 0, tp / jnp.where(den_r > 0, den_r, 1.0), 0.0)
    den_f = 2.0 * tp + fp + fn
    f1 = jnp.where(tp > 0, (2.0 * tp) / jnp.where(den_f > 0, den_f, 1.0), 0.0)
    lo = lax.broadcasted_iota(jnp.int32, (1, 128), 1)
    res_ref[...] = jnp.where(lo == 0, prec,
                             jnp.where(lo == 1, rec,
                                       jnp.where(lo == 2, f1, 0.0)))


def kernel(output, target):
    parts = _make_sc()(output, target)
    res = pl.pallas_call(
        _fin_body,
        out_shape=jax.ShapeDtypeStruct((1, 128), jnp.float32),
    )(parts)
    return res[0, :3]


# speedup vs baseline: 1473.5915x; 1.0915x over previous
import functools

import jax
import jax.numpy as jnp
from jax import lax
from jax.experimental import pallas as pl
from jax.experimental.pallas import tpu as pltpu
from jax.experimental.pallas import tpu_sc as plsc

_TH = 0.05
_TAU = 0.3
_LEN = 10

_N = 32768
_CHUNK = 8192
_NCHUNK = _N // _CHUNK
_NGRP = _CHUNK // 16
_MAXE = 16384
_MAXP = 3008


def _it16():
    return lax.broadcasted_iota(jnp.int32, (16,), 0)


def _sc_body(out_hbm, tgt_hbm, res_hbm, cb_f, cb_i, raw_b, ps, pe,
             rmax, ridx, cmax, cidx, t_taken, p_mut, v16, acc):
    wid = lax.axis_index("s") * 2 + lax.axis_index("c")

    def extract(row, hbm, cb, is_pred):
        def test(v):
            return (v >= _TH) if is_pred else (v != 0)

        ones = jnp.ones((16,), jnp.int32)
        zeros = jnp.zeros((16,), jnp.int32)
        it = _it16()

        def chunk_body(ch, carry):
            cnt_b, prevv = carry
            pltpu.sync_copy(hbm.at[row, pl.ds(ch * _CHUNK, _CHUNK)], cb)

            def grp_body(g, cnt_b):
                for u in range(4):
                    base = (g * 4 + u) * 16
                    v = cb[pl.ds(base, 16)]
                    m = test(v)
                    pv_raw = plsc.load_gather(
                        cb, [jnp.maximum(base - 1 + it, 0)])
                    pv_b = test(pv_raw)
                    if u == 0:
                        first = jnp.logical_and(
                            it == 0, jnp.full((16,), g, jnp.int32) == 0)
                        pv_b = jnp.where(first, prevv == 1, pv_b)
                    bm = m != pv_b
                    gpos = ch * _CHUNK + base + it
                    cs = plsc.cumsum(jnp.where(bm, ones, zeros))
                    plsc.store_scatter(raw_b, [cnt_b + cs - 1], gpos,
                                       mask=bm)
                    cnt_b = cnt_b + plsc.all_reduce_population_count(bm)
                return cnt_b

            cnt_b = lax.fori_loop(0, _NGRP // 4, grp_body, cnt_b)
            lastv = plsc.load_gather(cb, [jnp.full((16,), _CHUNK - 1,
                                                   jnp.int32)])
            prevv = jnp.where(test(lastv), ones, zeros)
            return cnt_b, prevv

        cnt_b, prevv = lax.fori_loop(
            0, _NCHUNK, chunk_body, (zeros, zeros))
        tail = jnp.logical_and(it == 0, prevv == 1)
        plsc.store_scatter(raw_b, [cnt_b],
                           jnp.full((16,), _N, jnp.int32), mask=tail)
        cnt_b = cnt_b + jnp.where(prevv == 1, ones, zeros)
        return jnp.max(cnt_b) >> 1

    def filter_preds(cnt):
        def body(g, a):
            it = _it16()
            i = g * 16 + it
            inb = i < cnt
            ic = jnp.minimum(i, jnp.maximum(cnt - 1, 0))
            s = plsc.load_gather(raw_b, [2 * ic])
            e = plsc.load_gather(raw_b, [2 * ic + 1])
            ok = jnp.logical_and(inb, (e - s) >= _LEN)
            cs = plsc.cumsum(jnp.where(ok, jnp.ones((16,), jnp.int32),
                                       jnp.zeros((16,), jnp.int32)))
            idx = a + cs - 1
            plsc.store_scatter(ps, [idx], s, mask=ok)
            plsc.store_scatter(pe, [idx], e, mask=ok)
            return a + plsc.all_reduce_population_count(ok)

        a = lax.fori_loop(0, (cnt + 15) // 16, body,
                          jnp.zeros((16,), jnp.int32))
        return jnp.max(a)

    def merge(get_a, get_b, n_a, n_b, omax, oidx):
        it = _it16()
        per = (n_a + 15) // 16
        lo = it * per
        hi = jnp.minimum(lo + per, n_a)
        amax = jnp.maximum(n_a - 1, 0)
        bmax = jnp.maximum(n_b - 1, 0)
        ps0, _ = get_a(jnp.minimum(lo, amax))

        def bs_body(_, c):
            lo_k, hi_k = c
            act = lo_k < hi_k
            mid = (lo_k + hi_k) >> 1
            _, tem = get_b(jnp.minimum(mid, bmax))
            goright = jnp.logical_and(act, tem <= ps0)
            lo_k = jnp.where(goright, mid + 1, lo_k)
            hi_k = jnp.where(jnp.logical_and(act, jnp.logical_not(goright)),
                             mid, hi_k)
            return lo_k, hi_k

        k0, _ = lax.fori_loop(0, 14, bs_body,
                              (jnp.zeros((16,), jnp.int32),
                               jnp.full((16,), n_b, jnp.int32)))

        _, pe_last = get_a(jnp.minimum(jnp.maximum(hi - 1, 0), amax))

        def bs2_body(_, c):
            lo_k, hi_k = c
            act = lo_k < hi_k
            mid = (lo_k + hi_k) >> 1
            bsm, _ = get_b(jnp.minimum(mid, bmax))
            goright = jnp.logical_and(act, bsm < pe_last)
            lo_k = jnp.where(goright, mid + 1, lo_k)
            hi_k = jnp.where(jnp.logical_and(act, jnp.logical_not(goright)),
                             mid, hi_k)
            return lo_k, hi_k

        kend, _ = lax.fori_loop(0, 14, bs2_body,
                                (jnp.zeros((16,), jnp.int32),
                                 jnp.full((16,), n_b, jnp.int32)))
        lane_iters = jnp.where(hi > lo,
                               (hi - lo) + jnp.maximum(kend - k0, 0),
                               jnp.zeros((16,), jnp.int32))
        bound = jnp.max(lane_iters)

        def step(_, c):
            j, k, bv, bk = c
            act = j < hi
            jj = jnp.minimum(j, amax)
            kk = jnp.minimum(k, bmax)
            asj, aej = get_a(jj)
            bsk, bek = get_b(kk)
            kin = k < jnp.full((16,), n_b, jnp.int32)
            inter = jnp.minimum(aej, bek) - jnp.maximum(asj, bsk)
            ov = jnp.logical_and(jnp.logical_and(act, kin), inter > 0)
            la = (aej - asj).astype(jnp.float32)
            lb = (bek - bsk).astype(jnp.float32)
            inf_ = inter.astype(jnp.float32)
            den = jnp.where(ov, la + lb - inf_, jnp.ones((16,), jnp.float32))
            iou = inf_ / den
            better = jnp.logical_and(ov, iou > bv)
            bv = jnp.where(better, iou, bv)
            bk = jnp.where(better, kk, bk)
            adv_j = jnp.logical_and(
                act, jnp.logical_or(jnp.logical_not(kin), aej <= bek))
            adv_k = jnp.logical_and(jnp.logical_and(act, kin), bek <= aej)
            plsc.store_scatter(omax, [jj], bv, mask=adv_j)
            plsc.store_scatter(oidx, [jj], bk, mask=adv_j)
            j = jnp.where(adv_j, j + 1, j)
            bv = jnp.where(adv_j, jnp.full((16,), -1.0, jnp.float32), bv)
            bk = jnp.where(adv_j, jnp.zeros((16,), jnp.int32), bk)
            k = jnp.where(adv_k, k + 1, k)
            return j, k, bv, bk

        lax.fori_loop(0, bound, step,
                      (lo, k0, jnp.full((16,), -1.0, jnp.float32),
                       jnp.zeros((16,), jnp.int32)))

    def match(n_a, n_b):
        it = _it16()
        amax = jnp.maximum(n_a - 1, 0)
        bmax = jnp.maximum(n_b - 1, 0)

        def zero_body(g, ref, n):
            def body(gg, _):
                i = gg * 16 + it
                plsc.store_scatter(ref, [jnp.minimum(i, n - 1)],
                                   jnp.zeros((16,), jnp.int32), mask=i < n)
                return 0
            return body

        lax.fori_loop(0, (n_b + 15) // 16, zero_body(0, t_taken, n_b), 0)
        lax.fori_loop(0, (n_a + 15) // 16, zero_body(0, p_mut, n_a), 0)

        def pass_a(g, tp):
            i = g * 16 + it
            inb = i < n_a
            ii = jnp.minimum(i, amax)
            rv = plsc.load_gather(rmax, [ii])
            rk = plsc.load_gather(ridx, [ii])
            cj = plsc.load_gather(cidx, [jnp.minimum(rk, bmax)])
            i2 = jnp.logical_and(jnp.logical_and(inb, rv >= _TAU), cj == i)
            plsc.store_scatter(p_mut, [ii], jnp.ones((16,), jnp.int32),
                               mask=i2)
            plsc.store_scatter(t_taken, [rk], jnp.ones((16,), jnp.int32),
                               mask=i2)
            return tp + jnp.sum(jnp.where(i2, jnp.ones((16,), jnp.int32),
                                          jnp.zeros((16,), jnp.int32)))

        tp = lax.fori_loop(0, (n_a + 15) // 16, pass_a, jnp.int32(0))

        def pass_b(g, one):
            i = g * 16 + it
            inb = i < n_a
            ii = jnp.minimum(i, amax)
            rv = plsc.load_gather(rmax, [ii])
            rk = plsc.load_gather(ridx, [ii])
            im = plsc.load_gather(p_mut, [ii])
            tk = plsc.load_gather(t_taken, [jnp.minimum(rk, bmax)])
            m2 = jnp.logical_and(jnp.logical_and(inb, rv >= _TAU), im == 0)
            keep = jnp.logical_and(m2, tk == 0)
            return one + jnp.sum(jnp.where(keep, jnp.ones((16,), jnp.int32),
                                           jnp.zeros((16,), jnp.int32)))

        one = lax.fori_loop(0, (n_a + 15) // 16, pass_b, jnp.int32(0))

        def pass_c(g, one):
            i = g * 16 + it
            inb = i < n_b
            ii = jnp.minimum(i, bmax)
            cv = plsc.load_gather(cmax, [ii])
            ck = plsc.load_gather(cidx, [ii])
            rj = plsc.load_gather(ridx, [jnp.minimum(ck, amax)])
            pm = plsc.load_gather(p_mut, [jnp.minimum(ck, amax)])
            i1 = jnp.logical_and(cv >= _TAU, rj == i)
            m1 = jnp.logical_and(jnp.logical_and(inb, cv >= _TAU),
                                 jnp.logical_not(i1))
            keep = jnp.logical_and(m1, pm == 0)
            return one + jnp.sum(jnp.where(keep, jnp.ones((16,), jnp.int32),
                                           jnp.zeros((16,), jnp.int32)))

        one = lax.fori_loop(0, (n_b + 15) // 16, pass_c, one)
        acc[0] = tp
        acc[1] = one

    tp_all = jnp.float32(0.0)
    fp_all = jnp.float32(0.0)
    fn_all = jnp.float32(0.0)
    for r in range(2):
        row = wid * 2 + r
        cnt_p = extract(row, out_hbm, cb_f, True)
        n_a = filter_preds(cnt_p)
        n_b = extract(row, tgt_hbm, cb_i, False)
        acc[0] = jnp.int32(0)
        acc[1] = jnp.int32(0)
        both = jnp.logical_and(n_a > 0, n_b > 0)

        get_p = lambda i: (plsc.load_gather(ps, [i]),
                           plsc.load_gather(pe, [i]))
        get_t = lambda i: (plsc.load_gather(raw_b, [2 * i]),
                           plsc.load_gather(raw_b, [2 * i + 1]))

        @pl.when(both)
        def _do():
            merge(get_p, get_t, n_a, n_b, rmax, ridx)
            merge(get_t, get_p, n_b, n_a, cmax, cidx)
            match(n_a, n_b)

        tp = acc[0].astype(jnp.float32)
        one = acc[1].astype(jnp.float32)
        matched = tp + one
        tp_all = tp_all + tp
        fp_all = fp_all + jnp.maximum(n_a.astype(jnp.float32) - matched, 0.0)
        fn_all = fn_all + jnp.maximum(n_b.astype(jnp.float32) - matched, 0.0)

    it = _it16()
    v16[...] = jnp.where(it == 0, jnp.full((16,), tp_all, jnp.float32),
                         jnp.where(it == 1, jnp.full((16,), fp_all, jnp.float32),
                                   jnp.where(it == 2, jnp.full((16,), fn_all, jnp.float32),
                                             jnp.zeros((16,), jnp.float32))))
    pltpu.sync_copy(v16, res_hbm.at[wid, pl.ds(0, 16)])


def _make_sc():
    mesh = plsc.VectorSubcoreMesh(core_axis_name="c", subcore_axis_name="s")
    return functools.partial(
        pl.kernel,
        mesh=mesh,
        out_type=jax.ShapeDtypeStruct((32, 128), jnp.float32),
        compiler_params=pltpu.CompilerParams(needs_layout_passes=False),
        scratch_types=[
            pltpu.VMEM((_CHUNK,), jnp.float32),
            pltpu.VMEM((_CHUNK,), jnp.int32),
            pltpu.VMEM((2 * _MAXE,), jnp.int32),
            pltpu.VMEM((_MAXP,), jnp.int32),
            pltpu.VMEM((_MAXP,), jnp.int32),
            pltpu.VMEM((_MAXP,), jnp.float32),
            pltpu.VMEM((_MAXP,), jnp.int32),
            pltpu.VMEM((_MAXE,), jnp.float32),
            pltpu.VMEM((_MAXE,), jnp.int32),
            pltpu.VMEM((_MAXE,), jnp.int32),
            pltpu.VMEM((_MAXP,), jnp.int32),
            pltpu.VMEM((16,), jnp.float32),
            pltpu.SMEM((4,), jnp.int32),
        ],
    )(_sc_body)


def _fin_body(parts_ref, res_ref):
    x = parts_ref[...]
    lane = lax.broadcasted_iota(jnp.int32, (32, 128), 1)
    zero = jnp.zeros((32, 128), jnp.float32)
    tp = jnp.sum(jnp.where(lane == 0, x, zero))
    fp = jnp.sum(jnp.where(lane == 1, x, zero))
    fn = jnp.sum(jnp.where(lane == 2, x, zero))
    den_p = tp + fp
    prec = jnp.where(den_p > 0, tp / jnp.where(den_p > 0, den_p, 1.0), 0.0)
    den_r = tp + fn
    rec = jnp.where(den_r > 0, tp / jnp.where(den_r > 0, den_r, 1.0), 0.0)
    den_f = 2.0 * tp + fp + fn
    f1 = jnp.where(tp > 0, (2.0 * tp) / jnp.where(den_f > 0, den_f, 1.0), 0.0)
    lo = lax.broadcasted_iota(jnp.int32, (1, 128), 1)
    res_ref[...] = jnp.where(lo == 0, prec,
                             jnp.where(lo == 1, rec,
                                       jnp.where(lo == 2, f1, 0.0)))


def kernel(output, target):
    parts = _make_sc()(output, target)
    res = pl.pallas_call(
        _fin_body,
        out_shape=jax.ShapeDtypeStruct((1, 128), jnp.float32),
    )(parts)
    return res[0, :3]


# speedup vs baseline: 1500.3068x; 1.0181x over previous
import functools

import jax
import jax.numpy as jnp
from jax import lax
from jax.experimental import pallas as pl
from jax.experimental.pallas import tpu as pltpu
from jax.experimental.pallas import tpu_sc as plsc

_TH = 0.05
_TAU = 0.3
_LEN = 10

_N = 32768
_CHUNK = 8192
_NCHUNK = _N // _CHUNK
_NGRP = _CHUNK // 16
_MAXE = 16384
_MAXP = 3008


def _it16():
    return lax.broadcasted_iota(jnp.int32, (16,), 0)


def _sc_body(out_hbm, tgt_hbm, res_hbm, cb_f, cb_i, raw_b, ps, pe,
             rmax, ridx, cmax, cidx, t_taken, p_mut, v16, acc):
    wid = lax.axis_index("s") * 2 + lax.axis_index("c")

    def extract(row, hbm, cb, is_pred):
        def test(v):
            return (v >= _TH) if is_pred else (v != 0)

        ones = jnp.ones((16,), jnp.int32)
        zeros = jnp.zeros((16,), jnp.int32)
        it = _it16()

        def chunk_body(ch, carry):
            cnt_b, prevv = carry
            pltpu.sync_copy(hbm.at[row, pl.ds(ch * _CHUNK, _CHUNK)], cb)

            def grp_body(g, cnt_b):
                for u in range(8):
                    base = (g * 8 + u) * 16
                    v = cb[pl.ds(base, 16)]
                    m = test(v)
                    pv_raw = plsc.load_gather(
                        cb, [jnp.maximum(base - 1 + it, 0)])
                    pv_b = test(pv_raw)
                    if u == 0:
                        first = jnp.logical_and(
                            it == 0, jnp.full((16,), g, jnp.int32) == 0)
                        pv_b = jnp.where(first, prevv == 1, pv_b)
                    bm = m != pv_b
                    gpos = ch * _CHUNK + base + it
                    cs = plsc.cumsum(jnp.where(bm, ones, zeros))
                    plsc.store_scatter(raw_b, [cnt_b + cs - 1], gpos,
                                       mask=bm)
                    cnt_b = cnt_b + plsc.all_reduce_population_count(bm)
                return cnt_b

            cnt_b = lax.fori_loop(0, _NGRP // 8, grp_body, cnt_b)
            lastv = plsc.load_gather(cb, [jnp.full((16,), _CHUNK - 1,
                                                   jnp.int32)])
            prevv = jnp.where(test(lastv), ones, zeros)
            return cnt_b, prevv

        cnt_b, prevv = lax.fori_loop(
            0, _NCHUNK, chunk_body, (zeros, zeros))
        tail = jnp.logical_and(it == 0, prevv == 1)
        plsc.store_scatter(raw_b, [cnt_b],
                           jnp.full((16,), _N, jnp.int32), mask=tail)
        cnt_b = cnt_b + jnp.where(prevv == 1, ones, zeros)
        return jnp.max(cnt_b) >> 1

    def filter_preds(cnt):
        def body(g, a):
            it = _it16()
            i = g * 16 + it
            inb = i < cnt
            ic = jnp.minimum(i, jnp.maximum(cnt - 1, 0))
            s = plsc.load_gather(raw_b, [2 * ic])
            e = plsc.load_gather(raw_b, [2 * ic + 1])
            ok = jnp.logical_and(inb, (e - s) >= _LEN)
            cs = plsc.cumsum(jnp.where(ok, jnp.ones((16,), jnp.int32),
                                       jnp.zeros((16,), jnp.int32)))
            idx = a + cs - 1
            plsc.store_scatter(ps, [idx], s, mask=ok)
            plsc.store_scatter(pe, [idx], e, mask=ok)
            return a + plsc.all_reduce_population_count(ok)

        a = lax.fori_loop(0, (cnt + 15) // 16, body,
                          jnp.zeros((16,), jnp.int32))
        return jnp.max(a)

    def merge(get_a, get_b, n_a, n_b, omax, oidx):
        it = _it16()
        per = (n_a + 15) // 16
        lo = it * per
        hi = jnp.minimum(lo + per, n_a)
        amax = jnp.maximum(n_a - 1, 0)
        bmax = jnp.maximum(n_b - 1, 0)
        ps0, _ = get_a(jnp.minimum(lo, amax))

        def bs_body(_, c):
            lo_k, hi_k = c
            act = lo_k < hi_k
            mid = (lo_k + hi_k) >> 1
            _, tem = get_b(jnp.minimum(mid, bmax))
            goright = jnp.logical_and(act, tem <= ps0)
            lo_k = jnp.where(goright, mid + 1, lo_k)
            hi_k = jnp.where(jnp.logical_and(act, jnp.logical_not(goright)),
                             mid, hi_k)
            return lo_k, hi_k

        k0, _ = lax.fori_loop(0, 14, bs_body,
                              (jnp.zeros((16,), jnp.int32),
                               jnp.full((16,), n_b, jnp.int32)))

        _, pe_last = get_a(jnp.minimum(jnp.maximum(hi - 1, 0), amax))

        def bs2_body(_, c):
            lo_k, hi_k = c
            act = lo_k < hi_k
            mid = (lo_k + hi_k) >> 1
            bsm, _ = get_b(jnp.minimum(mid, bmax))
            goright = jnp.logical_and(act, bsm < pe_last)
            lo_k = jnp.where(goright, mid + 1, lo_k)
            hi_k = jnp.where(jnp.logical_and(act, jnp.logical_not(goright)),
                             mid, hi_k)
            return lo_k, hi_k

        kend, _ = lax.fori_loop(0, 14, bs2_body,
                                (jnp.zeros((16,), jnp.int32),
                                 jnp.full((16,), n_b, jnp.int32)))
        lane_iters = jnp.where(hi > lo,
                               (hi - lo) + jnp.maximum(kend - k0, 0),
                               jnp.zeros((16,), jnp.int32))
        bound = jnp.max(lane_iters)

        def step(_, c):
            j, k, bv, bk = c
            act = j < hi
            jj = jnp.minimum(j, amax)
            kk = jnp.minimum(k, bmax)
            asj, aej = get_a(jj)
            bsk, bek = get_b(kk)
            kin = k < jnp.full((16,), n_b, jnp.int32)
            inter = jnp.minimum(aej, bek) - jnp.maximum(asj, bsk)
            ov = jnp.logical_and(jnp.logical_and(act, kin), inter > 0)
            la = (aej - asj).astype(jnp.float32)
            lb = (bek - bsk).astype(jnp.float32)
            inf_ = inter.astype(jnp.float32)
            den = jnp.where(ov, la + lb - inf_, jnp.ones((16,), jnp.float32))
            iou = inf_ / den
            better = jnp.logical_and(ov, iou > bv)
            bv = jnp.where(better, iou, bv)
            bk = jnp.where(better, kk, bk)
            adv_j = jnp.logical_and(
                act, jnp.logical_or(jnp.logical_not(kin), aej <= bek))
            adv_k = jnp.logical_and(jnp.logical_and(act, kin), bek <= aej)
            plsc.store_scatter(omax, [jj], bv, mask=adv_j)
            plsc.store_scatter(oidx, [jj], bk, mask=adv_j)
            j = jnp.where(adv_j, j + 1, j)
            bv = jnp.where(adv_j, jnp.full((16,), -1.0, jnp.float32), bv)
            bk = jnp.where(adv_j, jnp.zeros((16,), jnp.int32), bk)
            k = jnp.where(adv_k, k + 1, k)
            return j, k, bv, bk

        lax.fori_loop(0, bound, step,
                      (lo, k0, jnp.full((16,), -1.0, jnp.float32),
                       jnp.zeros((16,), jnp.int32)))

    def match(n_a, n_b):
        it = _it16()
        amax = jnp.maximum(n_a - 1, 0)
        bmax = jnp.maximum(n_b - 1, 0)

        def zero_body(g, ref, n):
            def body(gg, _):
                i = gg * 16 + it
                plsc.store_scatter(ref, [jnp.minimum(i, n - 1)],
                                   jnp.zeros((16,), jnp.int32), mask=i < n)
                return 0
            return body

        lax.fori_loop(0, (n_b + 15) // 16, zero_body(0, t_taken, n_b), 0)
        lax.fori_loop(0, (n_a + 15) // 16, zero_body(0, p_mut, n_a), 0)

        def pass_a(g, tp):
            i = g * 16 + it
            inb = i < n_a
            ii = jnp.minimum(i, amax)
            rv = plsc.load_gather(rmax, [ii])
            rk = plsc.load_gather(ridx, [ii])
            cj = plsc.load_gather(cidx, [jnp.minimum(rk, bmax)])
            i2 = jnp.logical_and(jnp.logical_and(inb, rv >= _TAU), cj == i)
            plsc.store_scatter(p_mut, [ii], jnp.ones((16,), jnp.int32),
                               mask=i2)
            plsc.store_scatter(t_taken, [rk], jnp.ones((16,), jnp.int32),
                               mask=i2)
            return tp + jnp.sum(jnp.where(i2, jnp.ones((16,), jnp.int32),
                                          jnp.zeros((16,), jnp.int32)))

        tp = lax.fori_loop(0, (n_a + 15) // 16, pass_a, jnp.int32(0))

        def pass_b(g, one):
            i = g * 16 + it
            inb = i < n_a
            ii = jnp.minimum(i, amax)
            rv = plsc.load_gather(rmax, [ii])
            rk = plsc.load_gather(ridx, [ii])
            im = plsc.load_gather(p_mut, [ii])
            tk = plsc.load_gather(t_taken, [jnp.minimum(rk, bmax)])
            m2 = jnp.logical_and(jnp.logical_and(inb, rv >= _TAU), im == 0)
            keep = jnp.logical_and(m2, tk == 0)
            return one + jnp.sum(jnp.where(keep, jnp.ones((16,), jnp.int32),
                                           jnp.zeros((16,), jnp.int32)))

        one = lax.fori_loop(0, (n_a + 15) // 16, pass_b, jnp.int32(0))

        def pass_c(g, one):
            i = g * 16 + it
            inb = i < n_b
            ii = jnp.minimum(i, bmax)
            cv = plsc.load_gather(cmax, [ii])
            ck = plsc.load_gather(cidx, [ii])
            rj = plsc.load_gather(ridx, [jnp.minimum(ck, amax)])
            pm = plsc.load_gather(p_mut, [jnp.minimum(ck, amax)])
            i1 = jnp.logical_and(cv >= _TAU, rj == i)
            m1 = jnp.logical_and(jnp.logical_and(inb, cv >= _TAU),
                                 jnp.logical_not(i1))
            keep = jnp.logical_and(m1, pm == 0)
            return one + jnp.sum(jnp.where(keep, jnp.ones((16,), jnp.int32),
                                           jnp.zeros((16,), jnp.int32)))

        one = lax.fori_loop(0, (n_b + 15) // 16, pass_c, one)
        acc[0] = tp
        acc[1] = one

    tp_all = jnp.float32(0.0)
    fp_all = jnp.float32(0.0)
    fn_all = jnp.float32(0.0)
    for r in range(2):
        row = wid * 2 + r
        cnt_p = extract(row, out_hbm, cb_f, True)
        n_a = filter_preds(cnt_p)
        n_b = extract(row, tgt_hbm, cb_i, False)
        acc[0] = jnp.int32(0)
        acc[1] = jnp.int32(0)
        both = jnp.logical_and(n_a > 0, n_b > 0)

        get_p = lambda i: (plsc.load_gather(ps, [i]),
                           plsc.load_gather(pe, [i]))
        get_t = lambda i: (plsc.load_gather(raw_b, [2 * i]),
                           plsc.load_gather(raw_b, [2 * i + 1]))

        @pl.when(both)
        def _do():
            merge(get_p, get_t, n_a, n_b, rmax, ridx)
            merge(get_t, get_p, n_b, n_a, cmax, cidx)
            match(n_a, n_b)

        tp = acc[0].astype(jnp.float32)
        one = acc[1].astype(jnp.float32)
        matched = tp + one
        tp_all = tp_all + tp
        fp_all = fp_all + jnp.maximum(n_a.astype(jnp.float32) - matched, 0.0)
        fn_all = fn_all + jnp.maximum(n_b.astype(jnp.float32) - matched, 0.0)

    it = _it16()
    v16[...] = jnp.where(it == 0, jnp.full((16,), tp_all, jnp.float32),
                         jnp.where(it == 1, jnp.full((16,), fp_all, jnp.float32),
                                   jnp.where(it == 2, jnp.full((16,), fn_all, jnp.float32),
                                             jnp.zeros((16,), jnp.float32))))
    pltpu.sync_copy(v16, res_hbm.at[wid, pl.ds(0, 16)])


def _make_sc():
    mesh = plsc.VectorSubcoreMesh(core_axis_name="c", subcore_axis_name="s")
    return functools.partial(
        pl.kernel,
        mesh=mesh,
        out_type=jax.ShapeDtypeStruct((32, 128), jnp.float32),
        compiler_params=pltpu.CompilerParams(needs_layout_passes=False),
        scratch_types=[
            pltpu.VMEM((_CHUNK,), jnp.float32),
            pltpu.VMEM((_CHUNK,), jnp.int32),
            pltpu.VMEM((2 * _MAXE,), jnp.int32),
            pltpu.VMEM((_MAXP,), jnp.int32),
            pltpu.VMEM((_MAXP,), jnp.int32),
            pltpu.VMEM((_MAXP,), jnp.float32),
            pltpu.VMEM((_MAXP,), jnp.int32),
            pltpu.VMEM((_MAXE,), jnp.float32),
            pltpu.VMEM((_MAXE,), jnp.int32),
            pltpu.VMEM((_MAXE,), jnp.int32),
            pltpu.VMEM((_MAXP,), jnp.int32),
            pltpu.VMEM((16,), jnp.float32),
            pltpu.SMEM((4,), jnp.int32),
        ],
    )(_sc_body)


def _fin_body(parts_ref, res_ref):
    x = parts_ref[...]
    lane = lax.broadcasted_iota(jnp.int32, (32, 128), 1)
    zero = jnp.zeros((32, 128), jnp.float32)
    tp = jnp.sum(jnp.where(lane == 0, x, zero))
    fp = jnp.sum(jnp.where(lane == 1, x, zero))
    fn = jnp.sum(jnp.where(lane == 2, x, zero))
    den_p = tp + fp
    prec = jnp.where(den_p > 0, tp / jnp.where(den_p > 0, den_p, 1.0), 0.0)
    den_r = tp + fn
    rec = jnp.where(den_r > 0, tp / jnp.where(den_r > 0, den_r, 1.0), 0.0)
    den_f = 2.0 * tp + fp + fn
    f1 = jnp.where(tp > 0, (2.0 * tp) / jnp.where(den_f > 0, den_f, 1.0), 0.0)
    lo = lax.broadcasted_iota(jnp.int32, (1, 128), 1)
    res_ref[...] = jnp.where(lo == 0, prec,
                             jnp.where(lo == 1, rec,
                                       jnp.where(lo == 2, f1, 0.0)))


def kernel(output, target):
    parts = _make_sc()(output, target)
    res = pl.pallas_call(
        _fin_body,
        out_shape=jax.ShapeDtypeStruct((1, 128), jnp.float32),
    )(parts)
    return res[0, :3]
